# baseline (speedup 1.0000x reference)
import functools

import jax
import jax.numpy as jnp
from jax import lax
from jax.experimental import pallas as pl
from jax.experimental.pallas import tpu as pltpu
from jax.experimental.pallas import tpu_sc as plsc

_NC = 2
_NS = 16
_LANES = 16
_NW = _NC * _NS
_KDIM = 8
_GCHUNK = 128


def _sc_gather_aug(table, idx, B, M_total, M):
    wpb = _NW // B
    mpw = M // wpb
    ngc = mpw // _GCHUNK
    mesh = plsc.VectorSubcoreMesh(core_axis_name="c", subcore_axis_name="s")

    @functools.partial(
        pl.kernel,
        mesh=mesh,
        out_type=jax.ShapeDtypeStruct((B, _KDIM, M), jnp.float32),
        scratch_types=[
            pltpu.VMEM((mpw,), jnp.int32),
            pltpu.VMEM((ngc, _GCHUNK), jnp.int32),
            pltpu.VMEM((mpw, _LANES), jnp.float32),
            pltpu.VMEM((_KDIM, mpw), jnp.float32),
            pltpu.SemaphoreType.DMA,
        ],
        compiler_params=pltpu.CompilerParams(
            needs_layout_passes=False, use_tc_tiling_on_sc=False
        ),
    )
    def sc_kernel(table_hbm, idx_hbm, out_hbm, idx_v, g_v, rows_v, outT_v, sem):
        wid = lax.axis_index("s") * _NC + lax.axis_index("c")
        b = wid % B
        base = (wid // B) * mpw
        pltpu.sync_copy(idx_hbm.at[pl.ds(base, mpw)], idx_v)
        lane = lax.iota(jnp.int32, _LANES)
        zeros = jnp.zeros((_LANES,), jnp.float32)
        for j in range(mpw // _LANES):
            r = j * _LANES
            g_v[r // _GCHUNK, pl.ds(r % _GCHUNK, _LANES)] = (
                idx_v[pl.ds(r, _LANES)] + b * M_total
            )
        handles = [
            pltpu.async_copy(
                table_hbm.at[g_v.at[p]],
                rows_v.at[pl.ds(p * _GCHUNK, _GCHUNK)],
                sem,
            )
            for p in range(ngc)
        ]
        for k in range(4, _KDIM):
            for j in range(mpw // _LANES):
                outT_v[k, pl.ds(j * _LANES, _LANES)] = zeros
        for h in handles:
            h.wait()
        for j in range(mpw // _LANES):
            m_idx = lane + j * _LANES
            comp = [
                plsc.load_gather(rows_v, [m_idx, jnp.full((_LANES,), k, jnp.int32)])
                for k in range(3)
            ]
            sl = pl.ds(j * _LANES, _LANES)
            for k in range(3):
                outT_v[k, sl] = -2.0 * comp[k]
            outT_v[3, sl] = (
                comp[0] * comp[0] + comp[1] * comp[1] + comp[2] * comp[2]
            )
        pltpu.sync_copy(outT_v, out_hbm.at[b, :, pl.ds(base, mpw)])

    return sc_kernel(table, idx)


def _tc_nn(verticesT, caug, B, N, M):
    NBLK = 2048
    iota_row = jnp.arange(M, dtype=jnp.float32).reshape(1, M)

    def body(v_ref, c_ref, i_ref, o_ref):
        vt = v_ref[0]
        cross2 = lax.dot_general(
            vt,
            c_ref[0, 0:3, :],
            (((0,), (0,)), ((), ())),
            preferred_element_type=jnp.float32,
        )
        nchunk = M // 128
        c2row = c_ref[0, 3:4, :]
        val = cross2[:, 0:128] + c2row[:, 0:128]
        cidf = jnp.zeros((NBLK, 128), jnp.float32)
        for j in range(1, nchunk):
            sj = cross2[:, j * 128:(j + 1) * 128] + c2row[:, j * 128:(j + 1) * 128]
            lt = sj < val
            val = jnp.minimum(val, sj)
            cidf = jnp.where(lt, jnp.float32(j), cidf)
        mf = cidf * 128.0 + i_ref[:, 0:128]
        mnv = jnp.min(val, axis=1, keepdims=True)
        nn_f = jnp.min(jnp.where(val == mnv, mf, jnp.float32(M)), axis=1)
        nn = nn_f.astype(jnp.int32)
        b = pl.program_id(0)
        brow = jnp.full((1, NBLK), b, jnp.int32)
        o_ref[0] = jnp.concatenate([brow, nn.reshape(1, NBLK)], axis=0)

    return pl.pallas_call(
        body,
        grid=(B, N // NBLK),
        in_specs=[
            pl.BlockSpec((1, 3, NBLK), lambda b, i: (b, 0, i)),
            pl.BlockSpec((1, _KDIM, M), lambda b, i: (b, 0, 0)),
            pl.BlockSpec((1, M), lambda b, i: (0, 0)),
        ],
        out_specs=pl.BlockSpec((1, 2, NBLK), lambda b, i: (b, 0, i)),
        out_shape=jax.ShapeDtypeStruct((B, 2, N), jnp.int32),
    )(verticesT, caug, iota_row)


def kernel(vertices, collider, collision_vertices):
    B, N, _ = vertices.shape
    M_total = collider.shape[1]
    M = collision_vertices.shape[0]
    idx = collision_vertices.astype(jnp.int32)
    table = jnp.pad(collider, ((0, 0), (0, 0), (0, _LANES - 3))).reshape(
        B * M_total, _LANES
    )
    caug = _sc_gather_aug(table, idx, B, M_total, M)
    verticesT = jnp.transpose(vertices, (0, 2, 1))
    out = _tc_nn(verticesT, caug, B, N, M)
    return jnp.transpose(out, (0, 2, 1))


# speedup vs baseline: 1.0385x; 1.0385x over previous
import functools

import jax
import jax.numpy as jnp
from jax import lax
from jax.experimental import pallas as pl
from jax.experimental.pallas import tpu as pltpu
from jax.experimental.pallas import tpu_sc as plsc

_NC = 2
_NS = 16
_LANES = 16
_NW = _NC * _NS
_KDIM = 8
_GCHUNK = 128


def _sc_gather_aug(table, idx, B, M_total, M):
    wpb = _NW // B
    mpw = M // wpb
    ngc = mpw // _GCHUNK
    mesh = plsc.VectorSubcoreMesh(core_axis_name="c", subcore_axis_name="s")

    @functools.partial(
        pl.kernel,
        mesh=mesh,
        out_type=jax.ShapeDtypeStruct((B, _KDIM, M), jnp.float32),
        scratch_types=[
            pltpu.VMEM((mpw,), jnp.int32),
            pltpu.VMEM((ngc, _GCHUNK), jnp.int32),
            pltpu.VMEM((mpw, _LANES), jnp.float32),
            pltpu.VMEM((_KDIM, mpw), jnp.float32),
            pltpu.SemaphoreType.DMA,
        ],
        compiler_params=pltpu.CompilerParams(
            needs_layout_passes=False, use_tc_tiling_on_sc=False
        ),
    )
    def sc_kernel(table_hbm, idx_hbm, out_hbm, idx_v, g_v, rows_v, outT_v, sem):
        wid = lax.axis_index("s") * _NC + lax.axis_index("c")
        b = wid % B
        base = (wid // B) * mpw
        pltpu.sync_copy(idx_hbm.at[pl.ds(base, mpw)], idx_v)
        lane = lax.iota(jnp.int32, _LANES)
        zeros = jnp.zeros((_LANES,), jnp.float32)
        for j in range(mpw // _LANES):
            r = j * _LANES
            g_v[r // _GCHUNK, pl.ds(r % _GCHUNK, _LANES)] = (
                idx_v[pl.ds(r, _LANES)] + b * M_total
            )
        handles = [
            pltpu.async_copy(
                table_hbm.at[g_v.at[p]],
                rows_v.at[pl.ds(p * _GCHUNK, _GCHUNK)],
                sem,
            )
            for p in range(ngc)
        ]
        for k in range(6, _KDIM):
            for j in range(mpw // _LANES):
                outT_v[k, pl.ds(j * _LANES, _LANES)] = zeros
        for h in handles:
            h.wait()
        def bf16_round(x):
            u = plsc.bitcast(x, jnp.uint32)
            r = (u + 0x7FFF + ((u >> 16) & 1)) & jnp.uint32(0xFFFF0000)
            return plsc.bitcast(r, jnp.float32)

        for j in range(mpw // _LANES):
            m_idx = lane + j * _LANES
            comp = [
                plsc.load_gather(rows_v, [m_idx, jnp.full((_LANES,), k, jnp.int32)])
                for k in range(3)
            ]
            sl = pl.ds(j * _LANES, _LANES)
            for k in range(3):
                outT_v[k, sl] = -2.0 * comp[k]
            c2 = comp[0] * comp[0] + comp[1] * comp[1] + comp[2] * comp[2]
            h = bf16_round(c2)
            r1 = c2 - h
            mmid = bf16_round(r1)
            low = bf16_round(r1 - mmid)
            outT_v[3, sl] = h
            outT_v[4, sl] = mmid
            outT_v[5, sl] = low
        pltpu.sync_copy(outT_v, out_hbm.at[b, :, pl.ds(base, mpw)])

    return sc_kernel(table, idx)


def _tc_nn(verticesT, caug, B, N, M):
    NBLK = 2048
    iota_row = jnp.arange(M, dtype=jnp.float32).reshape(1, M)

    def body(v_ref, c_ref, i_ref, o_ref):
        vt = v_ref[0]
        vt6 = jnp.concatenate([vt, jnp.ones((3, NBLK), jnp.float32)], axis=0)
        score_m = lax.dot_general(
            vt6,
            c_ref[0, 0:6, :],
            (((0,), (0,)), ((), ())),
            preferred_element_type=jnp.float32,
        )
        nchunk = M // 128
        val = score_m[:, 0:128]
        cidf = jnp.zeros((NBLK, 128), jnp.float32)
        for j in range(1, nchunk):
            sj = score_m[:, j * 128:(j + 1) * 128]
            lt = sj < val
            val = jnp.minimum(val, sj)
            cidf = jnp.where(lt, jnp.float32(j), cidf)
        mf = cidf * 128.0 + i_ref[:, 0:128]
        mnv = jnp.min(val, axis=1, keepdims=True)
        nn_f = jnp.min(jnp.where(val == mnv, mf, jnp.float32(M)), axis=1)
        nn = nn_f.astype(jnp.int32)
        b = pl.program_id(0)
        brow = jnp.full((1, NBLK), b, jnp.int32)
        o_ref[0] = jnp.concatenate([brow, nn.reshape(1, NBLK)], axis=0)

    return pl.pallas_call(
        body,
        grid=(B, N // NBLK),
        in_specs=[
            pl.BlockSpec((1, 3, NBLK), lambda b, i: (b, 0, i)),
            pl.BlockSpec((1, _KDIM, M), lambda b, i: (b, 0, 0)),
            pl.BlockSpec((1, M), lambda b, i: (0, 0)),
        ],
        out_specs=pl.BlockSpec((1, 2, NBLK), lambda b, i: (b, 0, i)),
        out_shape=jax.ShapeDtypeStruct((B, 2, N), jnp.int32),
    )(verticesT, caug, iota_row)


def kernel(vertices, collider, collision_vertices):
    B, N, _ = vertices.shape
    M_total = collider.shape[1]
    M = collision_vertices.shape[0]
    idx = collision_vertices.astype(jnp.int32)
    table = jnp.pad(collider, ((0, 0), (0, 0), (0, _LANES - 3))).reshape(
        B * M_total, _LANES
    )
    caug = _sc_gather_aug(table, idx, B, M_total, M)
    verticesT = jnp.transpose(vertices, (0, 2, 1))
    out = _tc_nn(verticesT, caug, B, N, M)
    return jnp.transpose(out, (0, 2, 1))
